# baseline (speedup 1.0000x reference)
import jax
import jax.numpy as jnp
from jax.experimental import pallas as pl
import jax.experimental.pallas.tpu as pltpu

N = 10000
IN_DIM = 128
HID = 64
OUT_DIM = 128

BM = 256
EPS = 1e-5


def _body(x_ref, inc_ref, ew0_ref, eb0_ref, eg_ref, eb_ref, ew1_ref,
          eb1_ref, cw_ref, dw0_ref, db0_ref, g_ref, b_ref, dw1_ref,
          db1_ref, out_ref, msg_ref):
    @pl.when(pl.program_id(0) == 0)
    def _encode():
        h = jnp.dot(x_ref[...], ew0_ref[...],
                    preferred_element_type=jnp.float32)
        h = jnp.maximum(h + eb0_ref[...], 0.0)
        m = jnp.mean(h, axis=-1, keepdims=True)
        v = jnp.mean((h - m) ** 2, axis=-1, keepdims=True)
        h = (h - m) * jax.lax.rsqrt(v + EPS) * eg_ref[...] + eb_ref[...]
        h = jnp.dot(h, ew1_ref[...], preferred_element_type=jnp.float32)
        h = jnp.maximum(h + eb1_ref[...], 0.0)
        msgk = jnp.dot(h, cw_ref[...], preferred_element_type=jnp.float32)
        ones_col = (jax.lax.broadcasted_iota(jnp.int32, (N, HID), 1)
                    == 0).astype(jnp.float32)
        msg_ref[...] = jnp.concatenate([msgk, ones_col], axis=1)

    inc = inc_ref[...]
    res = jnp.dot(inc, msg_ref[...], preferred_element_type=jnp.float32)
    agg = res[:, :HID] / res[:, HID:HID + 1]
    d = jnp.dot(agg, dw0_ref[...], preferred_element_type=jnp.float32)
    d = jnp.maximum(d + db0_ref[...], 0.0)
    m = jnp.mean(d, axis=-1, keepdims=True)
    v = jnp.mean((d - m) ** 2, axis=-1, keepdims=True)
    d = (d - m) * jax.lax.rsqrt(v + EPS) * g_ref[...] + b_ref[...]
    d = jnp.dot(d, dw1_ref[...], preferred_element_type=jnp.float32)
    out_ref[...] = jnp.maximum(d + db1_ref[...], 0.0)


def kernel(x, incidence, enc_w0, enc_b0, enc_ln_g, enc_ln_b, enc_w1, enc_b1,
           conv_w, dec_w0, dec_b0, dec_ln_g, dec_ln_b, dec_w1, dec_b1):
    f32 = jnp.float32
    row2 = lambda a: a.reshape(1, -1)
    full = lambda shape: pl.BlockSpec(shape, lambda *_: (0,) * len(shape))

    out = pl.pallas_call(
        _body,
        grid=(pl.cdiv(N, BM),),
        in_specs=[
            full((N, IN_DIM)),
            pl.BlockSpec((BM, N), lambda i: (i, 0)),
            full((IN_DIM, HID)),
            full((1, HID)),
            full((1, HID)),
            full((1, HID)),
            full((HID, HID)),
            full((1, HID)),
            full((HID, HID)),
            full((HID, HID)),
            full((1, HID)),
            full((1, HID)),
            full((1, HID)),
            full((HID, OUT_DIM)),
            full((1, OUT_DIM)),
        ],
        out_specs=pl.BlockSpec((BM, OUT_DIM), lambda i: (i, 0)),
        out_shape=jax.ShapeDtypeStruct((N, OUT_DIM), f32),
        scratch_shapes=[pltpu.VMEM((N, 2 * HID), f32)],
        compiler_params=pltpu.CompilerParams(
            dimension_semantics=("arbitrary",),
        ),
    )(x, incidence, enc_w0, row2(enc_b0), row2(enc_ln_g), row2(enc_ln_b),
      enc_w1, row2(enc_b1), conv_w, dec_w0, row2(dec_b0), row2(dec_ln_g),
      row2(dec_ln_b), dec_w1, row2(dec_b1))
    return out


# speedup vs baseline: 1.0412x; 1.0412x over previous
import jax
import jax.numpy as jnp
from jax.experimental import pallas as pl
import jax.experimental.pallas.tpu as pltpu

N = 10000
IN_DIM = 128
HID = 64
OUT_DIM = 128

BM = 400
EPS = 1e-5


def _body(x_ref, inc_ref, ew0_ref, eb0_ref, eg_ref, eb_ref, ew1_ref,
          eb1_ref, cw_ref, dw0_ref, db0_ref, g_ref, b_ref, dw1_ref,
          db1_ref, out_ref, msg_ref):
    @pl.when(pl.program_id(0) == 0)
    def _encode():
        h = jnp.dot(x_ref[...], ew0_ref[...],
                    preferred_element_type=jnp.float32)
        h = jnp.maximum(h + eb0_ref[...], 0.0)
        m = jnp.mean(h, axis=-1, keepdims=True)
        v = jnp.mean((h - m) ** 2, axis=-1, keepdims=True)
        h = (h - m) * jax.lax.rsqrt(v + EPS) * eg_ref[...] + eb_ref[...]
        h = jnp.dot(h, ew1_ref[...], preferred_element_type=jnp.float32)
        h = jnp.maximum(h + eb1_ref[...], 0.0)
        msgk = jnp.dot(h, cw_ref[...], preferred_element_type=jnp.float32)
        ones_col = (jax.lax.broadcasted_iota(jnp.int32, (N, HID), 1)
                    == 0).astype(jnp.float32)
        msg_ref[...] = jnp.concatenate([msgk, ones_col], axis=1)

    inc = inc_ref[...]
    res = jax.lax.dot(inc, msg_ref[...], precision=jax.lax.Precision.DEFAULT,
                      preferred_element_type=jnp.float32)
    agg = res[:, :HID] / res[:, HID:HID + 1]
    d = jnp.dot(agg, dw0_ref[...], preferred_element_type=jnp.float32)
    d = jnp.maximum(d + db0_ref[...], 0.0)
    m = jnp.mean(d, axis=-1, keepdims=True)
    v = jnp.mean((d - m) ** 2, axis=-1, keepdims=True)
    d = (d - m) * jax.lax.rsqrt(v + EPS) * g_ref[...] + b_ref[...]
    d = jnp.dot(d, dw1_ref[...], preferred_element_type=jnp.float32)
    out_ref[...] = jnp.maximum(d + db1_ref[...], 0.0)


def kernel(x, incidence, enc_w0, enc_b0, enc_ln_g, enc_ln_b, enc_w1, enc_b1,
           conv_w, dec_w0, dec_b0, dec_ln_g, dec_ln_b, dec_w1, dec_b1):
    f32 = jnp.float32
    row2 = lambda a: a.reshape(1, -1)
    full = lambda shape: pl.BlockSpec(shape, lambda *_: (0,) * len(shape))

    out = pl.pallas_call(
        _body,
        grid=(pl.cdiv(N, BM),),
        in_specs=[
            full((N, IN_DIM)),
            pl.BlockSpec((BM, N), lambda i: (i, 0)),
            full((IN_DIM, HID)),
            full((1, HID)),
            full((1, HID)),
            full((1, HID)),
            full((HID, HID)),
            full((1, HID)),
            full((HID, HID)),
            full((HID, HID)),
            full((1, HID)),
            full((1, HID)),
            full((1, HID)),
            full((HID, OUT_DIM)),
            full((1, OUT_DIM)),
        ],
        out_specs=pl.BlockSpec((BM, OUT_DIM), lambda i: (i, 0)),
        out_shape=jax.ShapeDtypeStruct((N, OUT_DIM), f32),
        scratch_shapes=[pltpu.VMEM((N, 2 * HID), f32)],
        compiler_params=pltpu.CompilerParams(
            dimension_semantics=("arbitrary",),
        ),
    )(x, incidence, enc_w0, row2(enc_b0), row2(enc_ln_g), row2(enc_ln_b),
      enc_w1, row2(enc_b1), conv_w, dec_w0, row2(dec_b0), row2(dec_ln_g),
      row2(dec_ln_b), dec_w1, row2(dec_b1))
    return out
